# baseline (speedup 1.0000x reference)
import functools

import jax
import jax.numpy as jnp
from jax import lax
from jax.experimental import pallas as pl
from jax.experimental.pallas import tpu as pltpu
from jax.experimental.pallas import tpu_sc as plsc

N = 10000
E = 320000
D = 128
G = 64

NC, NS = 2, 16
NW = NC * NS
CH = 128
NCH0 = 80
NCH1 = 80
NCHM = max(NCH0, NCH1)
E_PAD = NS * (NCH0 + NCH1) * CH
NPAD = 10240
RPT = NPAD // NS


def _sc_mesh():
    return plsc.VectorSubcoreMesh(core_axis_name="c", subcore_axis_name="s",
                                  num_cores=NC, num_subcores=NS)


def _deg_body(dst_hbm, ones_hbm, zeros_hbm, out_hbm, idx_v, ones_v, deg_sh):
    c = lax.axis_index("c")
    s = lax.axis_index("s")
    w = c * NS + s
    n_me = NCH0 if NCH0 == NCH1 else jnp.where(c == 0, NCH0, NCH1)
    pltpu.sync_copy(zeros_hbm, deg_sh.at[pl.ds(s * RPT, RPT)])
    pltpu.sync_copy(ones_hbm, ones_v)
    pltpu.sync_copy(dst_hbm.at[w], idx_v)
    plsc.subcore_barrier()

    @pl.loop(0, n_me)
    def _(j):
        pltpu.sync_copy(ones_v, deg_sh.at[idx_v.at[j]], add=True)

    plsc.subcore_barrier()
    pltpu.sync_copy(deg_sh.at[pl.ds(s * RPT, RPT)],
                    out_hbm.at[c, pl.ds(s * RPT, RPT)])


@functools.cache
def _deg_kernel():
    return pl.kernel(
        _deg_body,
        out_type=jax.ShapeDtypeStruct((NC, NPAD, D), jnp.float32),
        mesh=_sc_mesh(),
        scratch_types=[
            pltpu.VMEM((NCHM, CH), jnp.int32),
            pltpu.VMEM((CH, D), jnp.float32),
            pltpu.VMEM_SHARED((NPAD, D), jnp.float32),
        ],
    )


def _deg(dstp, onesD, zerosD):
    return _deg_kernel()(dstp, onesD, zerosD)


def _prop_body(g_hbm, src_hbm, dst_hbm, zeros_hbm, out_hbm,
               src_v, dst_v, rows_v, acc_sh, sem):
    c = lax.axis_index("c")
    s = lax.axis_index("s")
    w = c * NS + s
    n_me = NCH0 if NCH0 == NCH1 else jnp.where(c == 0, NCH0, NCH1)
    pltpu.sync_copy(zeros_hbm, acc_sh.at[pl.ds(s * RPT, RPT)])
    pltpu.sync_copy(src_hbm.at[w], src_v)
    pltpu.sync_copy(dst_hbm.at[w], dst_v)
    plsc.subcore_barrier()

    @pl.loop(0, n_me)
    def _(j):
        pltpu.async_copy(g_hbm.at[src_v.at[j]], rows_v, sem).wait()
        pltpu.sync_copy(rows_v, acc_sh.at[dst_v.at[j]], add=True)

    plsc.subcore_barrier()
    pltpu.sync_copy(acc_sh.at[pl.ds(s * RPT, RPT)],
                    out_hbm.at[c, pl.ds(s * RPT, RPT)])


@functools.cache
def _prop_kernel():
    return pl.kernel(
        _prop_body,
        out_type=jax.ShapeDtypeStruct((NC, NPAD, D), jnp.float32),
        mesh=_sc_mesh(),
        scratch_types=[
            pltpu.VMEM((NCHM, CH), jnp.int32),
            pltpu.VMEM((NCHM, CH), jnp.int32),
            pltpu.VMEM((CH, D), jnp.float32),
            pltpu.VMEM_SHARED((NPAD, D), jnp.float32),
            pltpu.SemaphoreType.DMA,
        ],
    )


def _prop(g, srcp, dstp, zerosD):
    return _prop_kernel()(g, srcp, dstp, zerosD)


_BLK = 1280


def _dinv_of(d0, d1):
    deg = d0[:, :1] + d1[:, :1] + 1.0
    return lax.rsqrt(jnp.maximum(deg, 1.0))


def _tc_a_body(x_ref, w_ref, d0_ref, d1_ref, o_ref):
    h = jnp.dot(x_ref[...], w_ref[...], preferred_element_type=jnp.float32)
    o_ref[...] = h * _dinv_of(d0_ref[...], d1_ref[...])


def _tc_a(x_pad, W1, degp0, degp1):
    grid = NPAD // _BLK
    return pl.pallas_call(
        _tc_a_body,
        grid=(grid,),
        in_specs=[
            pl.BlockSpec((_BLK, D), lambda i: (i, 0)),
            pl.BlockSpec((D, D), lambda i: (0, 0)),
            pl.BlockSpec((_BLK, 16), lambda i: (i, 0)),
            pl.BlockSpec((_BLK, 16), lambda i: (i, 0)),
        ],
        out_specs=pl.BlockSpec((_BLK, D), lambda i: (i, 0)),
        out_shape=jax.ShapeDtypeStruct((NPAD, D), jnp.float32),
    )(x_pad, W1, degp0, degp1)


def _tc_b_body(a0_ref, a1_ref, g_ref, d0_ref, d1_ref, b_ref, w_ref, o_ref):
    dinv = _dinv_of(d0_ref[...], d1_ref[...])
    pre = dinv * (a0_ref[...] + a1_ref[...] + g_ref[...]) + b_ref[...]
    h1 = jnp.maximum(pre, 0.0)
    h2 = jnp.dot(h1, w_ref[...], preferred_element_type=jnp.float32)
    o_ref[...] = h2 * dinv


def _tc_b(a0, a1, g1, degp0, degp1, b1, W2):
    grid = NPAD // _BLK
    return pl.pallas_call(
        _tc_b_body,
        grid=(grid,),
        in_specs=[
            pl.BlockSpec((_BLK, D), lambda i: (i, 0)),
            pl.BlockSpec((_BLK, D), lambda i: (i, 0)),
            pl.BlockSpec((_BLK, D), lambda i: (i, 0)),
            pl.BlockSpec((_BLK, 16), lambda i: (i, 0)),
            pl.BlockSpec((_BLK, 16), lambda i: (i, 0)),
            pl.BlockSpec((1, D), lambda i: (0, 0)),
            pl.BlockSpec((D, D), lambda i: (0, 0)),
        ],
        out_specs=pl.BlockSpec((_BLK, D), lambda i: (i, 0)),
        out_shape=jax.ShapeDtypeStruct((NPAD, D), jnp.float32),
    )(a0, a1, g1, degp0, degp1, b1, W2)


_CBLK = 1024
_NEG = -1e30


def _tc_c_body(a0_ref, a1_ref, g_ref, d0_ref, d1_ref, b_ref,
               batc_ref, batr_ref, lw_ref, lb_ref,
               xc_ref, out_ref, s0_ref, scnt_ref, smax_ref):
    i = pl.program_id(0)

    @pl.when(i == 0)
    def _init():
        s0_ref[...] = jnp.zeros((G, D), jnp.float32)
        scnt_ref[...] = jnp.zeros((G, D), jnp.float32)
        smax_ref[...] = jnp.full((G, D), _NEG, jnp.float32)

    dinv = _dinv_of(d0_ref[...], d1_ref[...])
    pre = dinv * (a0_ref[...] + a1_ref[...] + g_ref[...]) + b_ref[...]
    h = jnp.maximum(pre, 0.0)

    bat_row = batr_ref[...]
    gids = lax.broadcasted_iota(jnp.int32, (G, 1), 0)
    oh_t = (gids == bat_row).astype(jnp.float32)
    s0_ref[...] += jnp.dot(oh_t, h, preferred_element_type=jnp.float32)
    scnt_ref[...] += jnp.dot(oh_t, jnp.ones_like(h),
                             preferred_element_type=jnp.float32)

    bat_col = batc_ref[...]

    def _mx(gi, sm):
        masked = jnp.where(bat_col == gi, h, _NEG)
        bmax = jnp.max(masked, axis=0, keepdims=True)
        rowsel = gids == gi
        return jnp.where(rowsel, jnp.maximum(sm, bmax), sm)

    smax_ref[...] = lax.fori_loop(0, G, _mx, smax_ref[...])

    @pl.when(i == pl.num_programs(0) - 1)
    def _fin():
        s0 = s0_ref[...]
        cnt = jnp.maximum(scnt_ref[...], 1.0)
        x1 = s0 / cnt
        xc = jnp.concatenate([s0, x1, smax_ref[...]], axis=1)
        xc_ref[...] = xc
        out_ref[...] = (jnp.dot(xc, lw_ref[...],
                                preferred_element_type=jnp.float32)
                        + lb_ref[...])


def _tc_c(a0, a1, g2, degp0, degp1, b2, batch_col, batch_row, lw, lb):
    grid = NPAD // _CBLK
    return pl.pallas_call(
        _tc_c_body,
        grid=(grid,),
        in_specs=[
            pl.BlockSpec((_CBLK, D), lambda i: (i, 0)),
            pl.BlockSpec((_CBLK, D), lambda i: (i, 0)),
            pl.BlockSpec((_CBLK, D), lambda i: (i, 0)),
            pl.BlockSpec((_CBLK, 16), lambda i: (i, 0)),
            pl.BlockSpec((_CBLK, 16), lambda i: (i, 0)),
            pl.BlockSpec((1, D), lambda i: (0, 0)),
            pl.BlockSpec((_CBLK, 1), lambda i: (i, 0)),
            pl.BlockSpec((1, _CBLK), lambda i: (0, i)),
            pl.BlockSpec((3 * D, D), lambda i: (0, 0)),
            pl.BlockSpec((1, D), lambda i: (0, 0)),
        ],
        out_specs=[
            pl.BlockSpec((G, 3 * D), lambda i: (0, 0)),
            pl.BlockSpec((G, D), lambda i: (0, 0)),
        ],
        out_shape=[
            jax.ShapeDtypeStruct((G, 3 * D), jnp.float32),
            jax.ShapeDtypeStruct((G, D), jnp.float32),
        ],
        scratch_shapes=[
            pltpu.VMEM((G, D), jnp.float32),
            pltpu.VMEM((G, D), jnp.float32),
            pltpu.VMEM((G, D), jnp.float32),
        ],
        compiler_params=pltpu.CompilerParams(
            dimension_semantics=("arbitrary",)),
    )(a0, a1, g2, degp0, degp1, b2, batch_col, batch_row, lw, lb)


def _split_edges(idx):
    padded = jnp.concatenate(
        [idx, jnp.full((E_PAD - E,), N, jnp.int32)])
    e0 = NS * NCH0 * CH
    a = padded[:e0].reshape(NS, NCH0, CH)
    b = padded[e0:].reshape(NS, NCH1, CH)
    if NCH0 < NCHM:
        a = jnp.concatenate(
            [a, jnp.full((NS, NCHM - NCH0, CH), N, jnp.int32)], axis=1)
    if NCH1 < NCHM:
        b = jnp.concatenate(
            [b, jnp.full((NS, NCHM - NCH1, CH), N, jnp.int32)], axis=1)
    return jnp.concatenate([a, b], axis=0)


def kernel(x, edge_index, batch, W1, b1, W2, b2, lin_W, lin_b):
    f32 = jnp.float32
    srcp = _split_edges(edge_index[0])
    dstp = _split_edges(edge_index[1])
    x_pad = jnp.concatenate([x, jnp.zeros((NPAD - N, D), f32)], axis=0)
    bat_pad = jnp.concatenate(
        [batch.astype(jnp.int32), jnp.full((NPAD - N,), G, jnp.int32)])
    batch_col = bat_pad.reshape(NPAD, 1)
    batch_row = bat_pad.reshape(1, NPAD)
    onesD = jnp.ones((CH, D), f32)
    zerosD = jnp.zeros((RPT, D), f32)

    degp = _deg(dstp, onesD, zerosD)
    d0, d1 = degp[0, :, :16], degp[1, :, :16]
    g1 = _tc_a(x_pad, W1, d0, d1)
    acc1 = _prop(g1, srcp, dstp, zerosD)
    g2 = _tc_b(acc1[0], acc1[1], g1, d0, d1, b1.reshape(1, D), W2)
    acc2 = _prop(g2, srcp, dstp, zerosD)
    lw = jnp.zeros((3 * D, D), f32).at[:, :32].set(lin_W)
    lb = jnp.zeros((1, D), f32).at[0, :32].set(lin_b)
    xc, outp = _tc_c(acc2[0], acc2[1], g2, d0, d1, b2.reshape(1, D),
                     batch_col, batch_row, lw, lb)
    return (xc, outp[:, :32])


# speedup vs baseline: 2.6960x; 2.6960x over previous
import functools

import jax
import jax.numpy as jnp
from jax import lax
from jax.experimental import pallas as pl
from jax.experimental.pallas import tpu as pltpu
from jax.experimental.pallas import tpu_sc as plsc

N = 10000
E = 320000
D = 128
G = 64

NC, NS = 2, 16
NW = NC * NS
CH = 128
NCH0 = 80
NCH1 = 80
NCHM = max(NCH0, NCH1)
E_PAD = NS * (NCH0 + NCH1) * CH
NPAD = 10240
RPT = NPAD // NS


def _sc_mesh():
    return plsc.VectorSubcoreMesh(core_axis_name="c", subcore_axis_name="s",
                                  num_cores=NC, num_subcores=NS)


def _deg_body(dst_hbm, ones_hbm, zeros_hbm, out_hbm, idx_v, ones_v, deg_sh):
    c = lax.axis_index("c")
    s = lax.axis_index("s")
    w = c * NS + s
    n_me = NCH0 if NCH0 == NCH1 else jnp.where(c == 0, NCH0, NCH1)
    pltpu.sync_copy(zeros_hbm, deg_sh.at[pl.ds(s * RPT, RPT)])
    pltpu.sync_copy(ones_hbm, ones_v)
    pltpu.sync_copy(dst_hbm.at[w], idx_v)
    plsc.subcore_barrier()

    @pl.loop(0, n_me)
    def _(j):
        pltpu.sync_copy(ones_v, deg_sh.at[idx_v.at[j]], add=True)

    plsc.subcore_barrier()
    pltpu.sync_copy(deg_sh.at[pl.ds(s * RPT, RPT)],
                    out_hbm.at[c, pl.ds(s * RPT, RPT)])


@functools.cache
def _deg_kernel():
    return pl.kernel(
        _deg_body,
        out_type=jax.ShapeDtypeStruct((NC, NPAD, D), jnp.float32),
        mesh=_sc_mesh(),
        scratch_types=[
            pltpu.VMEM((NCHM, CH), jnp.int32),
            pltpu.VMEM((CH, D), jnp.float32),
            pltpu.VMEM_SHARED((NPAD, D), jnp.float32),
        ],
    )


def _deg(dstp, onesD, zerosD):
    return _deg_kernel()(dstp, onesD, zerosD)


def _prop_body(g_hbm, src_hbm, dst_hbm, zeros_hbm, out_hbm,
               src_v, dst_v, rows_v, acc_sh, sem):
    c = lax.axis_index("c")
    s = lax.axis_index("s")
    w = c * NS + s
    n_me = NCH0 if NCH0 == NCH1 else jnp.where(c == 0, NCH0, NCH1)
    pltpu.sync_copy(zeros_hbm, acc_sh.at[pl.ds(s * RPT, RPT)])
    pltpu.sync_copy(src_hbm.at[w], src_v)
    pltpu.sync_copy(dst_hbm.at[w], dst_v)
    plsc.subcore_barrier()

    @pl.loop(0, n_me)
    def _(j):
        pltpu.async_copy(g_hbm.at[src_v.at[j]], rows_v, sem).wait()
        pltpu.sync_copy(rows_v, acc_sh.at[dst_v.at[j]], add=True)

    plsc.subcore_barrier()
    pltpu.sync_copy(acc_sh.at[pl.ds(s * RPT, RPT)],
                    out_hbm.at[c, pl.ds(s * RPT, RPT)])


@functools.cache
def _prop_kernel():
    return pl.kernel(
        _prop_body,
        out_type=jax.ShapeDtypeStruct((NC, NPAD, D), jnp.float32),
        mesh=_sc_mesh(),
        scratch_types=[
            pltpu.VMEM((NCHM, CH), jnp.int32),
            pltpu.VMEM((NCHM, CH), jnp.int32),
            pltpu.VMEM((CH, D), jnp.float32),
            pltpu.VMEM_SHARED((NPAD, D), jnp.float32),
            pltpu.SemaphoreType.DMA,
        ],
    )


def _prop(g, srcp, dstp, zerosD):
    return _prop_kernel()(g, srcp, dstp, zerosD)


_BLK = 1280


def _dinv_of(d0, d1):
    deg = d0[:, :1] + d1[:, :1] + 1.0
    return lax.rsqrt(jnp.maximum(deg, 1.0))


def _tc_a_body(x_ref, w_ref, d0_ref, d1_ref, o_ref):
    h = jnp.dot(x_ref[...], w_ref[...], preferred_element_type=jnp.float32)
    o_ref[...] = h * _dinv_of(d0_ref[...], d1_ref[...])


def _tc_a(x_pad, W1, degp0, degp1):
    grid = NPAD // _BLK
    return pl.pallas_call(
        _tc_a_body,
        grid=(grid,),
        in_specs=[
            pl.BlockSpec((_BLK, D), lambda i: (i, 0)),
            pl.BlockSpec((D, D), lambda i: (0, 0)),
            pl.BlockSpec((_BLK, 16), lambda i: (i, 0)),
            pl.BlockSpec((_BLK, 16), lambda i: (i, 0)),
        ],
        out_specs=pl.BlockSpec((_BLK, D), lambda i: (i, 0)),
        out_shape=jax.ShapeDtypeStruct((NPAD, D), jnp.float32),
    )(x_pad, W1, degp0, degp1)


def _tc_b_body(a0_ref, a1_ref, g_ref, d0_ref, d1_ref, b_ref, w_ref, o_ref):
    dinv = _dinv_of(d0_ref[...], d1_ref[...])
    pre = dinv * (a0_ref[...] + a1_ref[...] + g_ref[...]) + b_ref[...]
    h1 = jnp.maximum(pre, 0.0)
    h2 = jnp.dot(h1, w_ref[...], preferred_element_type=jnp.float32)
    o_ref[...] = h2 * dinv


def _tc_b(a0, a1, g1, degp0, degp1, b1, W2):
    grid = NPAD // _BLK
    return pl.pallas_call(
        _tc_b_body,
        grid=(grid,),
        in_specs=[
            pl.BlockSpec((_BLK, D), lambda i: (i, 0)),
            pl.BlockSpec((_BLK, D), lambda i: (i, 0)),
            pl.BlockSpec((_BLK, D), lambda i: (i, 0)),
            pl.BlockSpec((_BLK, 16), lambda i: (i, 0)),
            pl.BlockSpec((_BLK, 16), lambda i: (i, 0)),
            pl.BlockSpec((1, D), lambda i: (0, 0)),
            pl.BlockSpec((D, D), lambda i: (0, 0)),
        ],
        out_specs=pl.BlockSpec((_BLK, D), lambda i: (i, 0)),
        out_shape=jax.ShapeDtypeStruct((NPAD, D), jnp.float32),
    )(a0, a1, g1, degp0, degp1, b1, W2)


_CBLK = 1024
_NEG = -1e30


def _tc_c_body(a0_ref, a1_ref, g_ref, d0_ref, d1_ref, b_ref,
               batc_ref, batr_ref, lw_ref, lb_ref,
               xc_ref, out_ref, s0_ref, scnt_ref, smax_ref):
    i = pl.program_id(0)

    @pl.when(i == 0)
    def _init():
        s0_ref[...] = jnp.zeros((G, D), jnp.float32)
        scnt_ref[...] = jnp.zeros((G, D), jnp.float32)
        smax_ref[...] = jnp.full((G, D), _NEG, jnp.float32)

    dinv = _dinv_of(d0_ref[...], d1_ref[...])
    pre = dinv * (a0_ref[...] + a1_ref[...] + g_ref[...]) + b_ref[...]
    h = jnp.maximum(pre, 0.0)

    bat_row = batr_ref[...]
    gids = lax.broadcasted_iota(jnp.int32, (G, 1), 0)
    oh_t = (gids == bat_row).astype(jnp.float32)
    s0_ref[...] += jnp.dot(oh_t, h, preferred_element_type=jnp.float32)
    scnt_ref[...] += jnp.dot(oh_t, jnp.ones_like(h),
                             preferred_element_type=jnp.float32)

    bat_col = batc_ref[...]

    def _mx(gi, sm):
        masked = jnp.where(bat_col == gi, h, _NEG)
        bmax = jnp.max(masked, axis=0, keepdims=True)
        rowsel = gids == gi
        return jnp.where(rowsel, jnp.maximum(sm, bmax), sm)

    glo = batc_ref[0, 0]
    ghi = jnp.minimum(batc_ref[_CBLK - 1, 0], G - 1)
    smax_ref[...] = lax.fori_loop(glo, ghi + 1, _mx, smax_ref[...])

    @pl.when(i == pl.num_programs(0) - 1)
    def _fin():
        s0 = s0_ref[...]
        cnt = jnp.maximum(scnt_ref[...], 1.0)
        x1 = s0 / cnt
        xc = jnp.concatenate([s0, x1, smax_ref[...]], axis=1)
        xc_ref[...] = xc
        out_ref[...] = (jnp.dot(xc, lw_ref[...],
                                preferred_element_type=jnp.float32)
                        + lb_ref[...])


def _tc_c(a0, a1, g2, degp0, degp1, b2, batch_col, batch_row, lw, lb):
    grid = NPAD // _CBLK
    return pl.pallas_call(
        _tc_c_body,
        grid=(grid,),
        in_specs=[
            pl.BlockSpec((_CBLK, D), lambda i: (i, 0)),
            pl.BlockSpec((_CBLK, D), lambda i: (i, 0)),
            pl.BlockSpec((_CBLK, D), lambda i: (i, 0)),
            pl.BlockSpec((_CBLK, 16), lambda i: (i, 0)),
            pl.BlockSpec((_CBLK, 16), lambda i: (i, 0)),
            pl.BlockSpec((1, D), lambda i: (0, 0)),
            pl.BlockSpec((_CBLK, 1), lambda i: (i, 0)),
            pl.BlockSpec((1, _CBLK), lambda i: (0, i)),
            pl.BlockSpec((3 * D, D), lambda i: (0, 0)),
            pl.BlockSpec((1, D), lambda i: (0, 0)),
        ],
        out_specs=[
            pl.BlockSpec((G, 3 * D), lambda i: (0, 0)),
            pl.BlockSpec((G, D), lambda i: (0, 0)),
        ],
        out_shape=[
            jax.ShapeDtypeStruct((G, 3 * D), jnp.float32),
            jax.ShapeDtypeStruct((G, D), jnp.float32),
        ],
        scratch_shapes=[
            pltpu.VMEM((G, D), jnp.float32),
            pltpu.VMEM((G, D), jnp.float32),
            pltpu.VMEM((G, D), jnp.float32),
        ],
        compiler_params=pltpu.CompilerParams(
            dimension_semantics=("arbitrary",)),
    )(a0, a1, g2, degp0, degp1, b2, batch_col, batch_row, lw, lb)


def _split_edges(idx):
    spread = N + jnp.arange(E_PAD - E, dtype=jnp.int32) % (NPAD - N)
    padded = jnp.concatenate([idx, spread])
    e0 = NS * NCH0 * CH
    a = padded[:e0].reshape(NS, NCH0, CH)
    b = padded[e0:].reshape(NS, NCH1, CH)
    if NCH0 < NCHM:
        a = jnp.concatenate(
            [a, jnp.full((NS, NCHM - NCH0, CH), N, jnp.int32)], axis=1)
    if NCH1 < NCHM:
        b = jnp.concatenate(
            [b, jnp.full((NS, NCHM - NCH1, CH), N, jnp.int32)], axis=1)
    return jnp.concatenate([a, b], axis=0)


def kernel(x, edge_index, batch, W1, b1, W2, b2, lin_W, lin_b):
    f32 = jnp.float32
    srcp = _split_edges(edge_index[0])
    dstp = _split_edges(edge_index[1])
    x_pad = jnp.concatenate([x, jnp.zeros((NPAD - N, D), f32)], axis=0)
    bat_pad = jnp.concatenate(
        [batch.astype(jnp.int32), jnp.full((NPAD - N,), G, jnp.int32)])
    batch_col = bat_pad.reshape(NPAD, 1)
    batch_row = bat_pad.reshape(1, NPAD)
    onesD = jnp.ones((CH, D), f32)
    zerosD = jnp.zeros((RPT, D), f32)

    degp = _deg(dstp, onesD, zerosD)
    d0, d1 = degp[0, :, :16], degp[1, :, :16]
    g1 = _tc_a(x_pad, W1, d0, d1)
    acc1 = _prop(g1, srcp, dstp, zerosD)
    g2 = _tc_b(acc1[0], acc1[1], g1, d0, d1, b1.reshape(1, D), W2)
    acc2 = _prop(g2, srcp, dstp, zerosD)
    lw = jnp.zeros((3 * D, D), f32).at[:, :32].set(lin_W)
    lb = jnp.zeros((1, D), f32).at[0, :32].set(lin_b)
    xc, outp = _tc_c(acc2[0], acc2[1], g2, d0, d1, b2.reshape(1, D),
                     batch_col, batch_row, lw, lb)
    return (xc, outp[:, :32])


# speedup vs baseline: 3.3852x; 1.2556x over previous
import functools

import jax
import jax.numpy as jnp
from jax import lax
from jax.experimental import pallas as pl
from jax.experimental.pallas import tpu as pltpu
from jax.experimental.pallas import tpu_sc as plsc

N = 10000
E = 320000
D = 128
G = 64

NC, NS = 2, 16
NW = NC * NS
CH = 128
NCH0 = 80
NCH1 = 80
NCHM = max(NCH0, NCH1)
E_PAD = NS * (NCH0 + NCH1) * CH
NPH = 5
PCH = NCH0 // NPH
NPAD = 10240
RPT = NPAD // NS


def _sc_mesh():
    return plsc.VectorSubcoreMesh(core_axis_name="c", subcore_axis_name="s",
                                  num_cores=NC, num_subcores=NS)


def _deg_body(dst_hbm, ones_hbm, zeros_hbm, out_hbm, idx_v, ones_v, deg_sh):
    c = lax.axis_index("c")
    s = lax.axis_index("s")
    w = c * NS + s
    n_me = NCH0 if NCH0 == NCH1 else jnp.where(c == 0, NCH0, NCH1)
    pltpu.sync_copy(zeros_hbm, deg_sh.at[pl.ds(s * RPT, RPT)])
    pltpu.sync_copy(ones_hbm, ones_v)
    pltpu.sync_copy(dst_hbm.at[w], idx_v)
    plsc.subcore_barrier()

    @pl.loop(0, n_me)
    def _(j):
        pltpu.sync_copy(ones_v, deg_sh.at[idx_v.at[j]], add=True)

    plsc.subcore_barrier()
    pltpu.sync_copy(deg_sh.at[pl.ds(s * RPT, RPT)],
                    out_hbm.at[c, pl.ds(s * RPT, RPT)])


@functools.cache
def _deg_kernel():
    return pl.kernel(
        _deg_body,
        out_type=jax.ShapeDtypeStruct((NC, NPAD, D), jnp.float32),
        mesh=_sc_mesh(),
        scratch_types=[
            pltpu.VMEM((NCHM, CH), jnp.int32),
            pltpu.VMEM((CH, D), jnp.float32),
            pltpu.VMEM_SHARED((NPAD, D), jnp.float32),
        ],
    )


def _deg(dstp, onesD, zerosD):
    return _deg_kernel()(dstp, onesD, zerosD)


def _prop_body(g_hbm, src_hbm, dst_hbm, zeros_hbm, out_hbm,
               src_v, dst_v, rows_a, rows_b, acc_sh, sem_a, sem_b):
    c = lax.axis_index("c")
    s = lax.axis_index("s")
    w = c * NS + s
    pltpu.sync_copy(zeros_hbm, acc_sh.at[pl.ds(s * RPT, RPT)])
    plsc.subcore_barrier()

    for p in range(NPH):
        pltpu.sync_copy(src_hbm.at[w, pl.ds(p * PCH, PCH)], src_v)
        pltpu.sync_copy(dst_hbm.at[w, pl.ds(p * PCH, PCH)], dst_v)
        pltpu.async_copy(g_hbm.at[src_v.at[0]], rows_a, sem_a)

        @pl.loop(0, PCH, step=2)
        def _(j):
            pltpu.async_copy(g_hbm.at[src_v.at[j + 1]], rows_b, sem_b)
            pltpu.make_async_copy(g_hbm.at[src_v.at[j]], rows_a, sem_a).wait()
            pltpu.sync_copy(rows_a, acc_sh.at[dst_v.at[j]], add=True)
            nxt = jnp.where(j + 2 < PCH, j + 2, 0)
            pltpu.async_copy(g_hbm.at[src_v.at[nxt]], rows_a, sem_a)
            pltpu.make_async_copy(g_hbm.at[src_v.at[j + 1]], rows_b, sem_b).wait()
            pltpu.sync_copy(rows_b, acc_sh.at[dst_v.at[j + 1]], add=True)

        pltpu.make_async_copy(g_hbm.at[src_v.at[0]], rows_a, sem_a).wait()

    plsc.subcore_barrier()
    pltpu.sync_copy(acc_sh.at[pl.ds(s * RPT, RPT)],
                    out_hbm.at[c, pl.ds(s * RPT, RPT)])


@functools.cache
def _prop_kernel():
    return pl.kernel(
        _prop_body,
        out_type=jax.ShapeDtypeStruct((NC, NPAD, D), jnp.float32),
        mesh=_sc_mesh(),
        scratch_types=[
            pltpu.VMEM((PCH, CH), jnp.int32),
            pltpu.VMEM((PCH, CH), jnp.int32),
            pltpu.VMEM((CH, D), jnp.float32),
            pltpu.VMEM((CH, D), jnp.float32),
            pltpu.VMEM_SHARED((NPAD, D), jnp.float32),
            pltpu.SemaphoreType.DMA,
            pltpu.SemaphoreType.DMA,
        ],
    )


def _prop(g, srcp, dstp, zerosD):
    return _prop_kernel()(g, srcp, dstp, zerosD)


_BLK = 1280


def _dinv_of(d0, d1):
    deg = d0[:, :1] + d1[:, :1] + 1.0
    return lax.rsqrt(jnp.maximum(deg, 1.0))


def _tc_a_body(x_ref, w_ref, d0_ref, d1_ref, o_ref):
    h = jnp.dot(x_ref[...], w_ref[...], preferred_element_type=jnp.float32)
    o_ref[...] = h * _dinv_of(d0_ref[...], d1_ref[...])


def _tc_a(x_pad, W1, degp0, degp1):
    grid = NPAD // _BLK
    return pl.pallas_call(
        _tc_a_body,
        grid=(grid,),
        in_specs=[
            pl.BlockSpec((_BLK, D), lambda i: (i, 0)),
            pl.BlockSpec((D, D), lambda i: (0, 0)),
            pl.BlockSpec((_BLK, 16), lambda i: (i, 0)),
            pl.BlockSpec((_BLK, 16), lambda i: (i, 0)),
        ],
        out_specs=pl.BlockSpec((_BLK, D), lambda i: (i, 0)),
        out_shape=jax.ShapeDtypeStruct((NPAD, D), jnp.float32),
    )(x_pad, W1, degp0, degp1)


def _tc_b_body(a0_ref, a1_ref, g_ref, d0_ref, d1_ref, b_ref, w_ref, o_ref):
    dinv = _dinv_of(d0_ref[...], d1_ref[...])
    pre = dinv * (a0_ref[...] + a1_ref[...] + g_ref[...]) + b_ref[...]
    h1 = jnp.maximum(pre, 0.0)
    h2 = jnp.dot(h1, w_ref[...], preferred_element_type=jnp.float32)
    o_ref[...] = h2 * dinv


def _tc_b(a0, a1, g1, degp0, degp1, b1, W2):
    grid = NPAD // _BLK
    return pl.pallas_call(
        _tc_b_body,
        grid=(grid,),
        in_specs=[
            pl.BlockSpec((_BLK, D), lambda i: (i, 0)),
            pl.BlockSpec((_BLK, D), lambda i: (i, 0)),
            pl.BlockSpec((_BLK, D), lambda i: (i, 0)),
            pl.BlockSpec((_BLK, 16), lambda i: (i, 0)),
            pl.BlockSpec((_BLK, 16), lambda i: (i, 0)),
            pl.BlockSpec((1, D), lambda i: (0, 0)),
            pl.BlockSpec((D, D), lambda i: (0, 0)),
        ],
        out_specs=pl.BlockSpec((_BLK, D), lambda i: (i, 0)),
        out_shape=jax.ShapeDtypeStruct((NPAD, D), jnp.float32),
    )(a0, a1, g1, degp0, degp1, b1, W2)


_CBLK = 1024
_NEG = -1e30


def _tc_c_body(a0_ref, a1_ref, g_ref, d0_ref, d1_ref, b_ref,
               batc_ref, batr_ref, lw_ref, lb_ref,
               xc_ref, out_ref, s0_ref, scnt_ref, smax_ref):
    i = pl.program_id(0)

    @pl.when(i == 0)
    def _init():
        s0_ref[...] = jnp.zeros((G, D), jnp.float32)
        scnt_ref[...] = jnp.zeros((G, D), jnp.float32)
        smax_ref[...] = jnp.full((G, D), _NEG, jnp.float32)

    dinv = _dinv_of(d0_ref[...], d1_ref[...])
    pre = dinv * (a0_ref[...] + a1_ref[...] + g_ref[...]) + b_ref[...]
    h = jnp.maximum(pre, 0.0)

    bat_row = batr_ref[...]
    gids = lax.broadcasted_iota(jnp.int32, (G, 1), 0)
    oh_t = (gids == bat_row).astype(jnp.float32)
    s0_ref[...] += jnp.dot(oh_t, h, preferred_element_type=jnp.float32)
    scnt_ref[...] += jnp.dot(oh_t, jnp.ones_like(h),
                             preferred_element_type=jnp.float32)

    bat_col = batc_ref[...]

    def _mx(gi, sm):
        masked = jnp.where(bat_col == gi, h, _NEG)
        bmax = jnp.max(masked, axis=0, keepdims=True)
        rowsel = gids == gi
        return jnp.where(rowsel, jnp.maximum(sm, bmax), sm)

    glo = batc_ref[0, 0]
    ghi = jnp.minimum(batc_ref[_CBLK - 1, 0], G - 1)
    smax_ref[...] = lax.fori_loop(glo, ghi + 1, _mx, smax_ref[...])

    @pl.when(i == pl.num_programs(0) - 1)
    def _fin():
        s0 = s0_ref[...]
        cnt = jnp.maximum(scnt_ref[...], 1.0)
        x1 = s0 / cnt
        xc = jnp.concatenate([s0, x1, smax_ref[...]], axis=1)
        xc_ref[...] = xc
        out_ref[...] = (jnp.dot(xc, lw_ref[...],
                                preferred_element_type=jnp.float32)
                        + lb_ref[...])


def _tc_c(a0, a1, g2, degp0, degp1, b2, batch_col, batch_row, lw, lb):
    grid = NPAD // _CBLK
    return pl.pallas_call(
        _tc_c_body,
        grid=(grid,),
        in_specs=[
            pl.BlockSpec((_CBLK, D), lambda i: (i, 0)),
            pl.BlockSpec((_CBLK, D), lambda i: (i, 0)),
            pl.BlockSpec((_CBLK, D), lambda i: (i, 0)),
            pl.BlockSpec((_CBLK, 16), lambda i: (i, 0)),
            pl.BlockSpec((_CBLK, 16), lambda i: (i, 0)),
            pl.BlockSpec((1, D), lambda i: (0, 0)),
            pl.BlockSpec((_CBLK, 1), lambda i: (i, 0)),
            pl.BlockSpec((1, _CBLK), lambda i: (0, i)),
            pl.BlockSpec((3 * D, D), lambda i: (0, 0)),
            pl.BlockSpec((1, D), lambda i: (0, 0)),
        ],
        out_specs=[
            pl.BlockSpec((G, 3 * D), lambda i: (0, 0)),
            pl.BlockSpec((G, D), lambda i: (0, 0)),
        ],
        out_shape=[
            jax.ShapeDtypeStruct((G, 3 * D), jnp.float32),
            jax.ShapeDtypeStruct((G, D), jnp.float32),
        ],
        scratch_shapes=[
            pltpu.VMEM((G, D), jnp.float32),
            pltpu.VMEM((G, D), jnp.float32),
            pltpu.VMEM((G, D), jnp.float32),
        ],
        compiler_params=pltpu.CompilerParams(
            dimension_semantics=("arbitrary",)),
    )(a0, a1, g2, degp0, degp1, b2, batch_col, batch_row, lw, lb)


def _split_edges(idx):
    spread = N + jnp.arange(E_PAD - E, dtype=jnp.int32) % (NPAD - N)
    padded = jnp.concatenate([idx, spread])
    e0 = NS * NCH0 * CH
    a = padded[:e0].reshape(NS, NCH0, CH)
    b = padded[e0:].reshape(NS, NCH1, CH)
    if NCH0 < NCHM:
        a = jnp.concatenate(
            [a, jnp.full((NS, NCHM - NCH0, CH), N, jnp.int32)], axis=1)
    if NCH1 < NCHM:
        b = jnp.concatenate(
            [b, jnp.full((NS, NCHM - NCH1, CH), N, jnp.int32)], axis=1)
    return jnp.concatenate([a, b], axis=0)


def kernel(x, edge_index, batch, W1, b1, W2, b2, lin_W, lin_b):
    f32 = jnp.float32
    srcp = _split_edges(edge_index[0])
    dstp = _split_edges(edge_index[1])
    x_pad = jnp.concatenate([x, jnp.zeros((NPAD - N, D), f32)], axis=0)
    bat_pad = jnp.concatenate(
        [batch.astype(jnp.int32), jnp.full((NPAD - N,), G, jnp.int32)])
    batch_col = bat_pad.reshape(NPAD, 1)
    batch_row = bat_pad.reshape(1, NPAD)
    onesD = jnp.ones((CH, D), f32)
    zerosD = jnp.zeros((RPT, D), f32)

    degp = _deg(dstp, onesD, zerosD)
    d0, d1 = degp[0, :, :16], degp[1, :, :16]
    g1 = _tc_a(x_pad, W1, d0, d1)
    acc1 = _prop(g1, srcp, dstp, zerosD)
    g2 = _tc_b(acc1[0], acc1[1], g1, d0, d1, b1.reshape(1, D), W2)
    acc2 = _prop(g2, srcp, dstp, zerosD)
    lw = jnp.zeros((3 * D, D), f32).at[:, :32].set(lin_W)
    lb = jnp.zeros((1, D), f32).at[0, :32].set(lin_b)
    xc, outp = _tc_c(acc2[0], acc2[1], g2, d0, d1, b2.reshape(1, D),
                     batch_col, batch_row, lw, lb)
    return (xc, outp[:, :32])
